# baseline (speedup 1.0000x reference)
import functools

import jax
import jax.numpy as jnp
from jax import lax
from jax.experimental import pallas as pl
from jax.experimental.pallas import tpu as pltpu
from jax.experimental.pallas import tpu_sc as plsc

K_NN = 32
Q = 1024
D_FEAT = 128
N_TRAIN = 100000
BKT = 128
NBK = 784
N_PAD = NBK * BKT
KT = 49
TW = 2048
BPT = TW // BKT
QT = 256

FBW = 32
FPB = BKT // FBW
NFB = N_PAD // FBW

NW = 32
QPW = Q // NW
CAP = 64
NCAND = CAP * FBW
TOPBIT = 0x80000000
PAD_FBK = NFB - 1


def _dist_kernel(a_ref, b_ref, a2_ref, b2_ref, d_ref, m_ref):
    prod = lax.dot_general(a_ref[...], b_ref[...], (((1,), (1,)), ((), ())),
                           preferred_element_type=jnp.float32)
    d = a2_ref[...] + b2_ref[...] - 2.0 * prod
    for ch in range(BPT):
        piece = d[:, ch * BKT:(ch + 1) * BKT]
        d_ref[:, ch] = piece.reshape(QT // 8, 8, BKT)
        for s in range(FPB):
            m_ref[0, :, ch * FPB + s:ch * FPB + s + 1] = jnp.min(
                piece[:, s * FBW:(s + 1) * FBW], axis=1, keepdims=True)


def _distances(a, b_pad, a2, b2_pad):
    return pl.pallas_call(
        _dist_kernel,
        grid=(Q // QT, KT),
        in_specs=[
            pl.BlockSpec((QT, D_FEAT), lambda qi, ki: (qi, 0)),
            pl.BlockSpec((TW, D_FEAT), lambda qi, ki: (ki, 0)),
            pl.BlockSpec((QT, 1), lambda qi, ki: (qi, 0)),
            pl.BlockSpec((1, TW), lambda qi, ki: (0, ki)),
        ],
        out_specs=[
            pl.BlockSpec((QT // 8, BPT, 8, BKT), lambda qi, ki: (qi, ki, 0, 0)),
            pl.BlockSpec((1, QT, BPT * FPB), lambda qi, ki: (ki, qi, 0)),
        ],
        out_shape=[
            jax.ShapeDtypeStruct((Q // 8, NBK, 8, BKT), jnp.float32),
            jax.ShapeDtypeStruct((KT, Q, BPT * FPB), jnp.float32),
        ],
    )(a, b_pad, a2, b2_pad)


def _monotone_u32(v):
    b = plsc.bitcast(v, jnp.uint32)
    neg = b >= jnp.uint32(TOPBIT)
    return jnp.where(neg, ~b, b | jnp.uint32(TOPBIT))


def _iota16():
    return lax.iota(jnp.int32, 16)


def _radix_pass(u_ref, nchunks, s, lo, k, hist_ref, cum_ref):
    su = s.astype(jnp.uint32)

    def _zero(i, _):
        hist_ref[pl.ds(i * 16, 16)] = jnp.zeros((16,), jnp.int32)
        return 0
    lax.fori_loop(0, 256, _zero, 0)

    ones = jnp.ones((16,), jnp.int32)
    iota = _iota16()

    def _scan(c, _):
        u = u_ref[pl.ds(c * 16, 16)]
        d = u - lo
        binv = (d >> su).astype(jnp.int32)
        inr = (u >= lo) & (binv < 256)
        binc = jnp.minimum(binv, 255)
        flat = binc * 16 + iota
        plsc.addupdate_scatter(hist_ref, [flat], ones, mask=inr)
        return 0
    lax.fori_loop(0, nchunks, _scan, 0)

    def _cum(c, carry):
        bins = c * 16 + iota
        acc = jnp.zeros((16,), jnp.int32)
        for l in range(16):
            acc = acc + plsc.load_gather(hist_ref, [bins * 16 + l])
        cc = plsc.cumsum(acc) + carry
        cum_ref[pl.ds(c * 16, 16)] = cc
        return jnp.max(cc)
    lax.fori_loop(0, 16, _cum, jnp.int32(0))

    def _bs(i, st):
        lo_b, hi_b = st
        mid = (lo_b + hi_b) // 2
        c = cum_ref[pl.ds(mid, 16)][0]
        return jnp.where(c >= k, lo_b, mid + 1), jnp.where(c >= k, mid, hi_b)
    lo_b, _ = lax.fori_loop(0, 8, _bs, (jnp.int32(0), jnp.int32(255)))
    n_below = jnp.where(
        lo_b > 0, cum_ref[pl.ds(jnp.maximum(lo_b - 1, 0), 16)][0], 0)
    n_thru = cum_ref[pl.ds(lo_b, 16)][0]
    return lo_b, n_below, n_thru


def _start_shift(u_min, u_max):
    s = jnp.int32(24)
    s = jnp.where((u_min >> 24) == (u_max >> 24), jnp.int32(16), s)
    s = jnp.where((u_min >> 16) == (u_max >> 16), jnp.int32(8), s)
    s = jnp.where((u_min >> 8) == (u_max >> 8), jnp.int32(0), s)
    base_bits = jnp.where(s >= 24, jnp.uint32(0),
                          (u_min >> (s.astype(jnp.uint32) + 8))
                          << (s.astype(jnp.uint32) + 8))
    return s, base_bits


def _sc_body(m_hbm, d_hbm, y_hbm, out_hbm,
             m_u, ucand, colmat, vals, hist, cum,
             idbuf, ridx, colbuf, eqbuf, m_row, ybuf, outbuf, sem):
    wid = lax.axis_index("s") * 2 + lax.axis_index("c")
    iota = _iota16()

    def _one_query(qi, _):
        q = wid * QPW + qi
        qh = q // 8
        ql = q % 8
        pltpu.sync_copy(m_hbm.at[q], m_row)

        def _cvt(c, mm):
            vmin, vmax = mm
            u = _monotone_u32(m_row[pl.ds(c * 16, 16)])
            m_u[pl.ds(c * 16, 16)] = u
            return jnp.minimum(vmin, u), jnp.maximum(vmax, u)
        vmin, vmax = lax.fori_loop(
            0, NFB // 16, _cvt,
            (jnp.full((16,), 0xFFFFFFFF, jnp.uint32),
             jnp.zeros((16,), jnp.uint32)))
        u_lo = jnp.min(vmin)
        u_hi = jnp.max(vmax)

        s0, lo0 = _start_shift(u_lo, u_hi)

        def _cond2(st):
            s, lo, nglb, u2, done = st
            return ~done

        def _step2(st):
            s, lo, nglb, u2, done = st
            b, nb, nt = _radix_pass(m_u, NFB // 16, s, lo, K_NN - nglb,
                                    hist, cum)
            su = s.astype(jnp.uint32)
            lo_new = lo + (b.astype(jnp.uint32) << su)
            cnt_le = nglb + nt
            ok = (cnt_le <= CAP) | (s == 0)
            u2_new = lo_new + (jnp.uint32(1) << su)
            return (s - 8, lo_new, nglb + nb,
                    jnp.where(ok, u2_new, u2), done | ok)

        _, _, _, U2, _ = lax.while_loop(
            _cond2, _step2,
            (s0, lo0, jnp.int32(0), jnp.uint32(0), jnp.bool_(False)))

        def _coll(c, cnt):
            u = m_u[pl.ds(c * 16, 16)]
            msk = u < U2
            n = jnp.sum(msk.astype(jnp.int32))

            @pl.when((n > 0) & (cnt < CAP))
            def _():
                plsc.store_compressed(idbuf.at[pl.ds(cnt, 16)],
                                      c * 16 + iota, mask=msk)
            return jnp.minimum(cnt + n, jnp.int32(CAP))
        ncoll = lax.fori_loop(0, NFB // 16, _coll, jnp.int32(0))

        def _mkidx(c, _):
            ids = idbuf[pl.ds(c * 16, 16)]
            valid = (c * 16 + iota) < ncoll
            bid = jnp.where(valid, jnp.clip(ids, 0, NFB - 1), PAD_FBK)
            ridx[pl.ds(c * 16, 16)] = (qh * NBK + (bid >> 2)) * 8 + ql
            return 0
        lax.fori_loop(0, CAP // 16, _mkidx, 0)
        pltpu.async_copy(d_hbm.at[ridx.at[...]], vals, sem)

        def _mkcol(j, _):
            base = idbuf[pl.ds(j, 16)][0] * FBW
            for sub in range(FBW // 16):
                colmat[pl.ds(j * FBW + sub * 16, 16)] = base + sub * 16 + iota
            return 0
        lax.fori_loop(0, ncoll, _mkcol, 0)

        pltpu.make_async_copy(d_hbm.at[ridx.at[...]], vals, sem).wait()

        nchunk = ncoll * (FBW // 16)

        def _cvt2(j, _):
            off = (idbuf[pl.ds(j, 16)][0] & 3) * FBW
            for sub in range(FBW // 16):
                ucand[pl.ds(j * FBW + sub * 16, 16)] = _monotone_u32(
                    vals[j, pl.ds(off + sub * 16, 16)])
            return 0
        lax.fori_loop(0, ncoll, _cvt2, 0)

        s0b, lo0b = _start_shift(u_lo, U2 - 1)

        def _cond5(st):
            s, lo, nglb, nlt, ndone = st
            return s >= 0

        def _step5(st):
            s, lo, nglb, nlt, _ = st
            b, nb, nt = _radix_pass(ucand, nchunk, s, lo, K_NN - nglb,
                                    hist, cum)
            su = s.astype(jnp.uint32)
            lo_new = lo + (b.astype(jnp.uint32) << su)
            return (s - 8, lo_new, nglb + nb, nglb + nb, 0)

        _, Ustar, _, n_lt, _ = lax.while_loop(
            _cond5, _step5,
            (s0b, lo0b, jnp.int32(0), jnp.int32(0), 0))

        maxi = jnp.full((16,), 0x7FFFFFFF, jnp.int32)
        for c in range(3):
            eqbuf[pl.ds(c * 16, 16)] = maxi

        def _coll5(c, st):
            clt, ceq = st
            u = ucand[pl.ds(c * 16, 16)]
            cols = colmat[pl.ds(c * 16, 16)]
            mlt = u < Ustar
            meq = u == Ustar
            nlt = jnp.sum(mlt.astype(jnp.int32))
            neq = jnp.sum(meq.astype(jnp.int32))

            @pl.when(nlt > 0)
            def _():
                plsc.store_compressed(colbuf.at[pl.ds(clt, 16)], cols,
                                      mask=mlt)

            @pl.when((neq > 0) & (ceq < 32))
            def _():
                plsc.store_compressed(eqbuf.at[pl.ds(ceq, 16)], cols,
                                      mask=meq)
            return clt + nlt, jnp.minimum(ceq + neq, jnp.int32(32))
        _, _ = lax.fori_loop(0, nchunk, _coll5,
                             (jnp.int32(0), jnp.int32(0)))

        needed = K_NN - n_lt
        e0 = lax.sort(eqbuf[pl.ds(0, 16)])
        e1 = lax.sort(eqbuf[pl.ds(16, 16)])
        r1 = lax.rev(e1, (0,))
        lo16 = lax.sort(jnp.minimum(e0, r1))
        hi16 = lax.sort(jnp.maximum(e0, r1))

        @pl.when(needed > 0)
        def _():
            m0 = iota < needed
            plsc.store_compressed(colbuf.at[pl.ds(n_lt, 16)], lo16, mask=m0)

        @pl.when(needed > 16)
        def _():
            m1 = iota < (needed - 16)
            plsc.store_compressed(colbuf.at[pl.ds(n_lt + 16, 16)], hi16,
                                  mask=m1)

        pltpu.async_copy(y_hbm.at[colbuf.at[pl.ds(0, K_NN)]], ybuf,
                         sem).wait()
        tot = jnp.sum(ybuf[pl.ds(0, 16)]) + jnp.sum(ybuf[pl.ds(16, 16)])
        mean = jnp.full((16,), tot * (1.0 / K_NN), jnp.float32)
        plsc.store_scatter(outbuf, [jnp.full((16,), qi, jnp.int32)], mean,
                           mask=iota == 0)
        return 0

    lax.fori_loop(0, QPW, _one_query, 0)
    pltpu.sync_copy(outbuf, out_hbm.at[pl.ds(wid * QPW, QPW)])


def _sc_select(m, d2, y):
    mesh = plsc.VectorSubcoreMesh(core_axis_name="c", subcore_axis_name="s",
                                  num_cores=2, num_subcores=16)
    fn = pl.kernel(
        _sc_body,
        out_type=jax.ShapeDtypeStruct((Q,), jnp.float32),
        mesh=mesh,
        compiler_params=pltpu.CompilerParams(needs_layout_passes=False),
        scratch_types=[
            pltpu.VMEM((NFB,), jnp.uint32),
            pltpu.VMEM((NCAND,), jnp.uint32),
            pltpu.VMEM((NCAND,), jnp.int32),
            pltpu.VMEM((CAP, BKT), jnp.float32),
            pltpu.VMEM((4096,), jnp.int32),
            pltpu.VMEM((272,), jnp.int32),
            pltpu.VMEM((96,), jnp.int32),
            pltpu.VMEM((CAP,), jnp.int32),
            pltpu.VMEM((48,), jnp.int32),
            pltpu.VMEM((48,), jnp.int32),
            pltpu.VMEM((NFB,), jnp.float32),
            pltpu.VMEM((K_NN,), jnp.float32),
            pltpu.VMEM((QPW,), jnp.float32),
            pltpu.SemaphoreType.DMA,
        ],
    )
    return fn(m, d2, y)


def kernel(inputs, X_train, y_train):
    a = inputs.astype(jnp.float32)
    b = X_train.astype(jnp.float32)
    y = y_train.astype(jnp.float32)

    a2 = jnp.sum(a * a, axis=1, keepdims=True)
    b2 = jnp.sum(b * b, axis=1)
    b_pad = jnp.pad(b, ((0, N_PAD - N_TRAIN), (0, 0)))
    b2_pad = jnp.pad(b2, (0, N_PAD - N_TRAIN),
                     constant_values=jnp.float32(1e30))[None, :]

    d4, m3 = _distances(a, b_pad, a2, b2_pad)
    d2 = d4.reshape((Q // 8) * NBK * 8, BKT)
    m = m3.transpose(1, 0, 2).reshape(Q, NFB)
    return _sc_select(m, d2, y)


# speedup vs baseline: 1.1733x; 1.1733x over previous
import functools

import jax
import jax.numpy as jnp
from jax import lax
from jax.experimental import pallas as pl
from jax.experimental.pallas import tpu as pltpu
from jax.experimental.pallas import tpu_sc as plsc

K_NN = 32
Q = 1024
D_FEAT = 128
N_TRAIN = 100000
BKT = 128
NBK = 784
N_PAD = NBK * BKT
KT = 49
TW = 2048
BPT = TW // BKT
QT = 256

FBW = 32
FPB = BKT // FBW
NFB = N_PAD // FBW

NW = 32
QPW = Q // NW
CAP = 64
NCAND = CAP * FBW
TOPBIT = 0x80000000
PAD_FBK = NFB - 1


def _dist_kernel(a_ref, b_ref, a2_ref, b2_ref, d_ref, m_ref):
    prod = lax.dot_general(a_ref[...], b_ref[...], (((1,), (1,)), ((), ())),
                           preferred_element_type=jnp.float32)
    d = a2_ref[...] + b2_ref[...] - 2.0 * prod
    for ch in range(BPT):
        piece = d[:, ch * BKT:(ch + 1) * BKT]
        d_ref[:, ch] = piece.reshape(QT // 8, 8, BKT)
        for s in range(FPB):
            m_ref[0, :, ch * FPB + s:ch * FPB + s + 1] = jnp.min(
                piece[:, s * FBW:(s + 1) * FBW], axis=1, keepdims=True)


def _distances(a, b_pad, a2, b2_pad):
    return pl.pallas_call(
        _dist_kernel,
        grid=(Q // QT, KT),
        in_specs=[
            pl.BlockSpec((QT, D_FEAT), lambda qi, ki: (qi, 0)),
            pl.BlockSpec((TW, D_FEAT), lambda qi, ki: (ki, 0)),
            pl.BlockSpec((QT, 1), lambda qi, ki: (qi, 0)),
            pl.BlockSpec((1, TW), lambda qi, ki: (0, ki)),
        ],
        out_specs=[
            pl.BlockSpec((QT // 8, BPT, 8, BKT), lambda qi, ki: (qi, ki, 0, 0)),
            pl.BlockSpec((1, QT, BPT * FPB), lambda qi, ki: (ki, qi, 0)),
        ],
        out_shape=[
            jax.ShapeDtypeStruct((Q // 8, NBK, 8, BKT), jnp.float32),
            jax.ShapeDtypeStruct((KT, Q, BPT * FPB), jnp.float32),
        ],
    )(a, b_pad, a2, b2_pad)


def _monotone_u32(v):
    b = plsc.bitcast(v, jnp.uint32)
    neg = b >= jnp.uint32(TOPBIT)
    return jnp.where(neg, ~b, b | jnp.uint32(TOPBIT))


def _iota16():
    return lax.iota(jnp.int32, 16)


def _radix_pass(u_ref, nchunks, s, lo, k, hist_ref, cum_ref):
    su = s.astype(jnp.uint32)

    def _zero(i, _):
        hist_ref[pl.ds(i * 16, 16)] = jnp.zeros((16,), jnp.int32)
        return 0
    lax.fori_loop(0, 16, _zero, 0)

    ones = jnp.ones((16,), jnp.int32)

    def _scan(c, _):
        u = u_ref[pl.ds(c * 16, 16)]
        d = u - lo
        binv = d >> su
        inr = (u >= lo) & (binv < jnp.uint32(256))
        binc = jnp.minimum(binv, jnp.uint32(255)).astype(jnp.int32)
        plsc.addupdate_scatter(hist_ref, [binc], ones, mask=inr)
        return 0
    lax.fori_loop(0, nchunks, _scan, 0)

    def _cum(c, carry):
        cc = plsc.cumsum(hist_ref[pl.ds(c * 16, 16)]) + carry
        cum_ref[pl.ds(c * 16, 16)] = cc
        return jnp.max(cc)
    lax.fori_loop(0, 16, _cum, jnp.int32(0))

    def _bs(i, st):
        lo_b, hi_b = st
        mid = (lo_b + hi_b) // 2
        c = cum_ref[pl.ds(mid, 16)][0]
        return jnp.where(c >= k, lo_b, mid + 1), jnp.where(c >= k, mid, hi_b)
    lo_b, _ = lax.fori_loop(0, 8, _bs, (jnp.int32(0), jnp.int32(255)))
    n_below = jnp.where(
        lo_b > 0, cum_ref[pl.ds(jnp.maximum(lo_b - 1, 0), 16)][0], 0)
    n_thru = cum_ref[pl.ds(lo_b, 16)][0]
    return lo_b, n_below, n_thru


def _start_shift(u_min, u_max):
    s = jnp.int32(24)
    s = jnp.where((u_min >> 24) == (u_max >> 24), jnp.int32(16), s)
    s = jnp.where((u_min >> 16) == (u_max >> 16), jnp.int32(8), s)
    s = jnp.where((u_min >> 8) == (u_max >> 8), jnp.int32(0), s)
    base_bits = jnp.where(s >= 24, jnp.uint32(0),
                          (u_min >> (s.astype(jnp.uint32) + 8))
                          << (s.astype(jnp.uint32) + 8))
    return s, base_bits


def _sc_body(m_hbm, d_hbm, y_hbm, out_hbm,
             m_u, ucand, colmat, vals, hist, cum,
             idbuf, ridx, colbuf, eqbuf, m_row, ybuf, outbuf, sem):
    wid = lax.axis_index("s") * 2 + lax.axis_index("c")
    iota = _iota16()

    def _one_query(qi, _):
        q = wid * QPW + qi
        qh = q // 8
        ql = q % 8
        pltpu.sync_copy(m_hbm.at[q], m_row)

        def _cvt(c, mm):
            vmin, vmax = mm
            u = _monotone_u32(m_row[pl.ds(c * 16, 16)])
            m_u[pl.ds(c * 16, 16)] = u
            return jnp.minimum(vmin, u), jnp.maximum(vmax, u)
        vmin, vmax = lax.fori_loop(
            0, NFB // 16, _cvt,
            (jnp.full((16,), 0xFFFFFFFF, jnp.uint32),
             jnp.zeros((16,), jnp.uint32)))
        u_lo = jnp.min(vmin)
        u_hi = jnp.max(vmax)

        s0, lo0 = _start_shift(u_lo, u_hi)

        def _cond2(st):
            s, lo, nglb, u2, done = st
            return ~done

        def _step2(st):
            s, lo, nglb, u2, done = st
            b, nb, nt = _radix_pass(m_u, NFB // 16, s, lo, K_NN - nglb,
                                    hist, cum)
            su = s.astype(jnp.uint32)
            lo_new = lo + (b.astype(jnp.uint32) << su)
            cnt_le = nglb + nt
            ok = (cnt_le <= CAP) | (s == 0)
            u2_new = lo_new + (jnp.uint32(1) << su)
            return (s - 8, lo_new, nglb + nb,
                    jnp.where(ok, u2_new, u2), done | ok)

        _, _, _, U2, _ = lax.while_loop(
            _cond2, _step2,
            (s0, lo0, jnp.int32(0), jnp.uint32(0), jnp.bool_(False)))

        def _coll(c, cnt):
            u = m_u[pl.ds(c * 16, 16)]
            msk = u < U2
            n = jnp.sum(msk.astype(jnp.int32))

            @pl.when((n > 0) & (cnt < CAP))
            def _():
                plsc.store_compressed(idbuf.at[pl.ds(cnt, 16)],
                                      c * 16 + iota, mask=msk)
            return jnp.minimum(cnt + n, jnp.int32(CAP))
        ncoll = lax.fori_loop(0, NFB // 16, _coll, jnp.int32(0))

        def _mkidx(c, _):
            ids = idbuf[pl.ds(c * 16, 16)]
            valid = (c * 16 + iota) < ncoll
            bid = jnp.where(valid, jnp.clip(ids, 0, NFB - 1), PAD_FBK)
            ridx[pl.ds(c * 16, 16)] = (qh * NBK + (bid >> 2)) * 8 + ql
            return 0
        lax.fori_loop(0, CAP // 16, _mkidx, 0)
        pltpu.async_copy(d_hbm.at[ridx.at[...]], vals, sem)

        def _mkcol(j, _):
            base = idbuf[pl.ds(j, 16)][0] * FBW
            for sub in range(FBW // 16):
                colmat[pl.ds(j * FBW + sub * 16, 16)] = base + sub * 16 + iota
            return 0
        lax.fori_loop(0, ncoll, _mkcol, 0)

        pltpu.make_async_copy(d_hbm.at[ridx.at[...]], vals, sem).wait()

        nchunk = ncoll * (FBW // 16)

        def _cvt2(j, _):
            off = (idbuf[pl.ds(j, 16)][0] & 3) * FBW
            for sub in range(FBW // 16):
                ucand[pl.ds(j * FBW + sub * 16, 16)] = _monotone_u32(
                    vals[j, pl.ds(off + sub * 16, 16)])
            return 0
        lax.fori_loop(0, ncoll, _cvt2, 0)

        s0b, lo0b = _start_shift(u_lo, U2 - 1)

        def _cond5(st):
            s, lo, nglb, nlt, ndone = st
            return s >= 0

        def _step5(st):
            s, lo, nglb, nlt, _ = st
            b, nb, nt = _radix_pass(ucand, nchunk, s, lo, K_NN - nglb,
                                    hist, cum)
            su = s.astype(jnp.uint32)
            lo_new = lo + (b.astype(jnp.uint32) << su)
            return (s - 8, lo_new, nglb + nb, nglb + nb, 0)

        _, Ustar, _, n_lt, _ = lax.while_loop(
            _cond5, _step5,
            (s0b, lo0b, jnp.int32(0), jnp.int32(0), 0))

        maxi = jnp.full((16,), 0x7FFFFFFF, jnp.int32)
        for c in range(3):
            eqbuf[pl.ds(c * 16, 16)] = maxi

        def _coll5(c, st):
            clt, ceq = st
            u = ucand[pl.ds(c * 16, 16)]
            cols = colmat[pl.ds(c * 16, 16)]
            mlt = u < Ustar
            meq = u == Ustar
            nlt = jnp.sum(mlt.astype(jnp.int32))
            neq = jnp.sum(meq.astype(jnp.int32))

            @pl.when(nlt > 0)
            def _():
                plsc.store_compressed(colbuf.at[pl.ds(clt, 16)], cols,
                                      mask=mlt)

            @pl.when((neq > 0) & (ceq < 32))
            def _():
                plsc.store_compressed(eqbuf.at[pl.ds(ceq, 16)], cols,
                                      mask=meq)
            return clt + nlt, jnp.minimum(ceq + neq, jnp.int32(32))
        _, _ = lax.fori_loop(0, nchunk, _coll5,
                             (jnp.int32(0), jnp.int32(0)))

        needed = K_NN - n_lt
        e0 = lax.sort(eqbuf[pl.ds(0, 16)])
        e1 = lax.sort(eqbuf[pl.ds(16, 16)])
        r1 = lax.rev(e1, (0,))
        lo16 = lax.sort(jnp.minimum(e0, r1))
        hi16 = lax.sort(jnp.maximum(e0, r1))

        @pl.when(needed > 0)
        def _():
            m0 = iota < needed
            plsc.store_compressed(colbuf.at[pl.ds(n_lt, 16)], lo16, mask=m0)

        @pl.when(needed > 16)
        def _():
            m1 = iota < (needed - 16)
            plsc.store_compressed(colbuf.at[pl.ds(n_lt + 16, 16)], hi16,
                                  mask=m1)

        pltpu.async_copy(y_hbm.at[colbuf.at[pl.ds(0, K_NN)]], ybuf,
                         sem).wait()
        tot = jnp.sum(ybuf[pl.ds(0, 16)]) + jnp.sum(ybuf[pl.ds(16, 16)])
        mean = jnp.full((16,), tot * (1.0 / K_NN), jnp.float32)
        plsc.store_scatter(outbuf, [jnp.full((16,), qi, jnp.int32)], mean,
                           mask=iota == 0)
        return 0

    lax.fori_loop(0, QPW, _one_query, 0)
    pltpu.sync_copy(outbuf, out_hbm.at[pl.ds(wid * QPW, QPW)])


def _sc_select(m, d2, y):
    mesh = plsc.VectorSubcoreMesh(core_axis_name="c", subcore_axis_name="s",
                                  num_cores=2, num_subcores=16)
    fn = pl.kernel(
        _sc_body,
        out_type=jax.ShapeDtypeStruct((Q,), jnp.float32),
        mesh=mesh,
        compiler_params=pltpu.CompilerParams(needs_layout_passes=False),
        scratch_types=[
            pltpu.VMEM((NFB,), jnp.uint32),
            pltpu.VMEM((NCAND,), jnp.uint32),
            pltpu.VMEM((NCAND,), jnp.int32),
            pltpu.VMEM((CAP, BKT), jnp.float32),
            pltpu.VMEM((272,), jnp.int32),
            pltpu.VMEM((272,), jnp.int32),
            pltpu.VMEM((96,), jnp.int32),
            pltpu.VMEM((CAP,), jnp.int32),
            pltpu.VMEM((48,), jnp.int32),
            pltpu.VMEM((48,), jnp.int32),
            pltpu.VMEM((NFB,), jnp.float32),
            pltpu.VMEM((K_NN,), jnp.float32),
            pltpu.VMEM((QPW,), jnp.float32),
            pltpu.SemaphoreType.DMA,
        ],
    )
    return fn(m, d2, y)


def kernel(inputs, X_train, y_train):
    a = inputs.astype(jnp.float32)
    b = X_train.astype(jnp.float32)
    y = y_train.astype(jnp.float32)

    a2 = jnp.sum(a * a, axis=1, keepdims=True)
    b2 = jnp.sum(b * b, axis=1)
    b_pad = jnp.pad(b, ((0, N_PAD - N_TRAIN), (0, 0)))
    b2_pad = jnp.pad(b2, (0, N_PAD - N_TRAIN),
                     constant_values=jnp.float32(1e30))[None, :]

    d4, m3 = _distances(a, b_pad, a2, b2_pad)
    d2 = d4.reshape((Q // 8) * NBK * 8, BKT)
    m = m3.transpose(1, 0, 2).reshape(Q, NFB)
    return _sc_select(m, d2, y)
